# speedup vs baseline: 1.1142x; 1.1142x over previous
import functools

import jax
import jax.numpy as jnp
from jax import lax
from jax.experimental import pallas as pl
from jax.experimental.pallas import tpu as pltpu
from jax.experimental.pallas import tpu_sc as plsc

NC, NS = 2, 16
NW = NC * NS
CHUNK = 64


def _sc_gather_two(row_table, col_table, row_idx, col_idx):
  n = row_idx.shape[0]
  d = row_table.shape[1]
  per_w = n // NW
  steps = per_w // CHUNK
  mesh = plsc.VectorSubcoreMesh(core_axis_name="c", subcore_axis_name="s")
  out_sds = jax.ShapeDtypeStruct((n, d), row_table.dtype)

  @functools.partial(
      pl.kernel,
      out_type=(out_sds, out_sds),
      mesh=mesh,
      scratch_types=[
          pltpu.VMEM((CHUNK,), jnp.int32),
          pltpu.VMEM((CHUNK, d), row_table.dtype),
          pltpu.SemaphoreType.DMA,
      ],
  )
  def k(rt_hbm, ct_hbm, ri_hbm, ci_hbm, gr_hbm, gc_hbm, idx_v, rows_v, sem):
    wid = lax.axis_index("s") * NC + lax.axis_index("c")
    base = wid * per_w
    for c in range(steps):
      off = base + c * CHUNK
      pltpu.sync_copy(ri_hbm.at[pl.ds(off, CHUNK)], idx_v)
      pltpu.async_copy(rt_hbm.at[idx_v], rows_v, sem).wait()
      pltpu.sync_copy(rows_v, gr_hbm.at[pl.ds(off, CHUNK)])
      pltpu.sync_copy(ci_hbm.at[pl.ds(off, CHUNK)], idx_v)
      pltpu.async_copy(ct_hbm.at[idx_v], rows_v, sem).wait()
      pltpu.sync_copy(rows_v, gc_hbm.at[pl.ds(off, CHUNK)])

  return k(row_table, col_table, row_idx, col_idx)


def _tc_body(fp_ref, w_ref, b_ref, gr_ref, gc_ref, out_ref):
  p = fp_ref[...].astype(jnp.bfloat16)
  w = w_ref[...].astype(jnp.bfloat16)
  acc = jnp.dot(p, w, preferred_element_type=jnp.float32)
  out_ref[...] = acc + b_ref[...] + gr_ref[...] + gc_ref[...]


def _tc_project_add(fp2, w_pad, b2, g_row, g_col, block_rows=1024):
  n, pw = fp2.shape
  h = w_pad.shape[1]
  grid = (n // block_rows,)
  return pl.pallas_call(
      _tc_body,
      grid=grid,
      in_specs=[
          pl.BlockSpec((block_rows, pw), lambda i: (i, 0)),
          pl.BlockSpec((pw, h), lambda i: (0, 0)),
          pl.BlockSpec((1, h), lambda i: (0, 0)),
          pl.BlockSpec((block_rows, h), lambda i: (i, 0)),
          pl.BlockSpec((block_rows, h), lambda i: (i, 0)),
      ],
      out_specs=pl.BlockSpec((block_rows, h), lambda i: (i, 0)),
      out_shape=jax.ShapeDtypeStruct((n, h), jnp.float32),
  )(fp2, w_pad, b2, g_row, g_col)


def kernel(flattened_patches, W, b, row_table, col_table):
  bsz, s, pw = flattened_patches.shape
  h = W.shape[1]
  n = bsz * s
  fp2 = flattened_patches.reshape(n, pw)
  idx = fp2[:, :2].astype(jnp.int32)
  g_row, g_col = _sc_gather_two(row_table, col_table, idx[:, 0], idx[:, 1])
  w_pad = jnp.concatenate([jnp.zeros((2, h), W.dtype), W], axis=0)
  out2 = _tc_project_add(fp2, w_pad, b.reshape(1, h), g_row, g_col)
  return out2.reshape(bsz, s, h)
